# baseline (speedup 1.0000x reference)
import numpy as np
import jax
import jax.numpy as jnp
from jax import lax
from jax.experimental import pallas as pl
from jax.experimental.pallas import tpu as pltpu
from jax.experimental.pallas import tpu_sc as plsc

_B = 256
_LMAX = 256
_M = 2 * _LMAX - 1
_K = _LMAX * _LMAX
_NC, _NS, _L = 2, 16, 16
_NW = _NC * _NS


def _cost_partition():
    cols = np.arange(_M)
    starts = np.abs(cols - (_LMAX - 1))
    l_of_k = np.concatenate([np.arange(s, _LMAX) for s in starts])
    c_of_k = np.repeat(cols, _LMAX - starts)

    def spans_for(target):
        spans, ks = [], 0
        while ks < _K and len(spans) < _NW:
            c0 = int(c_of_k[ks])
            ke = ks + 16
            while ke < _K:
                nxt = min(ke + 16, _K)
                w = int(c_of_k[nxt - 1]) - c0 + 1
                if 256 * w + (nxt - ks) > target:
                    break
                ke = nxt
            spans.append((ks, ke))
            ks = ke
        return spans, ks

    lo, hi = 4096, 32768
    while lo < hi:
        mid = (lo + hi) // 2
        _, ks = spans_for(mid)
        if ks >= _K:
            hi = mid
        else:
            lo = mid + 1
    spans, ks = spans_for(lo)
    assert ks == _K and len(spans) <= _NW
    while len(spans) < _NW:
        i = max(range(len(spans)), key=lambda j: spans[j][1] - spans[j][0])
        a, b = spans[i]
        m = a + ((b - a) // 32) * 16
        spans[i:i + 1] = [(a, m), (m, b)]
    return spans, l_of_k, c_of_k


def _build_geometry():
    spans, l_of_k, c_of_k = _cost_partition()
    raw = []
    for ks, ke in spans:
        c0, c1 = int(c_of_k[ks]), int(c_of_k[ke - 1])
        raw.append((ks, ke - ks, c0, c1 - c0 + 1))

    order = sorted(range(_NW), key=lambda s: (raw[s][1], raw[s][3]))
    classes = []
    group = []
    for s in order:
        group.append(s)
        if len(group) == 4 or s == order[-1]:
            ln_c = max(raw[g][1] for g in group)
            mems = []
            w_need = 0
            for g in group:
                k0 = min(raw[g][0], _K - ln_c)
                c0 = int(c_of_k[k0])
                c1 = int(c_of_k[k0 + ln_c - 1])
                w_need = max(w_need, c1 - c0 + 1)
                mems.append((g, k0, c0))
            w_c = w_need
            offs = [(g, k0, min(c0, _M - w_c)) for g, k0, c0 in mems]
            classes.append((w_c, ln_c, offs))
            group = []

    ln_max = max(ln for _, ln, _ in classes)
    packed = np.zeros((_NW, ln_max), np.int32)
    for w_c, ln_c, offs in classes:
        for s, k0, c0c in offs:
            lk = l_of_k[k0:k0 + ln_c]
            ck = c_of_k[k0:k0 + ln_c]
            assert c0c >= 0 and ck.max() < c0c + w_c and ck.min() >= c0c
            packed[s, :ln_c] = (((ck - c0c).astype(np.int32) << 16)
                                | lk.astype(np.int32))
    return classes, packed, ln_max


_CLASSES, _PACKED, _LN_MAX = _build_geometry()


def _sc_body(tensor_hbm, idx_hbm, out_hbm, idx_v, ob0, ob1, isem0, isem1,
             osem0, osem1):
    wid = lax.axis_index("c") * _NS + lax.axis_index("s")
    pltpu.sync_copy(idx_hbm.at[wid], idx_v)

    for w_c, ln_c, offs in _CLASSES:
        if len(offs) == 1:
            s, k0v, c0c = offs[0]
            is_member = wid == s
        else:
            is_member = jnp.bool_(False)
            c0c = jnp.int32(0)
            k0v = jnp.int32(0)
            for s, k0, c0 in offs:
                hit = wid == s
                is_member = jnp.logical_or(is_member, hit)
                c0c = jnp.where(hit, jnp.int32(c0), c0c)
                k0v = jnp.where(hit, jnp.int32(k0), k0v)
            k0v = pl.multiple_of(k0v, 16)

        @pl.when(is_member)
        def _cls(w_c=w_c, ln_c=ln_c, c0c=c0c, k0v=k0v):
            ng = ln_c // _L

            def gather(slab, ob):
                @plsc.parallel_loop(0, ng, unroll=16)
                def _g(g):
                    iv = idx_v[pl.ds(g * _L, _L)]
                    rows = lax.shift_right_logical(iv, 16)
                    cls_ = lax.bitwise_and(iv, jnp.int32(0xFFFF))
                    ob[pl.ds(g * _L, _L)] = plsc.load_gather(
                        slab, [rows, cls_])

            def scoped(slab0, slab1):
                def in_copy(b, slab, sem):
                    return pltpu.make_async_copy(
                        tensor_hbm.at[b, pl.ds(c0c, w_c), :], slab, sem)

                def out_copy(b, ob, sem):
                    return pltpu.make_async_copy(
                        ob.at[pl.ds(0, ln_c)],
                        out_hbm.at[b, pl.ds(k0v, ln_c)], sem)

                in_copy(0, slab0, isem0).start()
                in_copy(1, slab1, isem1).start()

                @pl.loop(0, _B // 2)
                def _bb(bb):
                    b0 = bb * 2
                    b1 = b0 + 1

                    @pl.when(bb > 0)
                    def _():
                        out_copy(b0 - 2, ob0, osem0).wait()
                    in_copy(b0, slab0, isem0).wait()
                    gather(slab0, ob0)

                    @pl.when(bb < _B // 2 - 1)
                    def _():
                        in_copy(b0 + 2, slab0, isem0).start()
                    out_copy(b0, ob0, osem0).start()

                    @pl.when(bb > 0)
                    def _():
                        out_copy(b1 - 2, ob1, osem1).wait()
                    in_copy(b1, slab1, isem1).wait()
                    gather(slab1, ob1)

                    @pl.when(bb < _B // 2 - 1)
                    def _():
                        in_copy(b1 + 2, slab1, isem1).start()
                    out_copy(b1, ob1, osem1).start()

                out_copy(_B - 2, ob0, osem0).wait()
                out_copy(_B - 1, ob1, osem1).wait()

            pl.run_scoped(
                scoped,
                pltpu.VMEM((w_c, _LMAX), jnp.float32),
                pltpu.VMEM((w_c, _LMAX), jnp.float32),
            )


def kernel(tensor):
    tensor_t = jnp.swapaxes(tensor, 1, 2)
    idx = jnp.asarray(_PACKED)
    mesh = plsc.VectorSubcoreMesh(core_axis_name="c", subcore_axis_name="s")
    f = pl.kernel(
        _sc_body,
        out_type=jax.ShapeDtypeStruct((_B, _K), jnp.float32),
        mesh=mesh,
        compiler_params=pltpu.CompilerParams(
            use_tc_tiling_on_sc=False, needs_layout_passes=False),
        scratch_types=[
            pltpu.VMEM((_LN_MAX,), jnp.int32),
            pltpu.VMEM((_LN_MAX,), jnp.float32),
            pltpu.VMEM((_LN_MAX,), jnp.float32),
            pltpu.SemaphoreType.DMA,
            pltpu.SemaphoreType.DMA,
            pltpu.SemaphoreType.DMA,
            pltpu.SemaphoreType.DMA,
        ],
    )
    return f(tensor_t, idx)


# speedup vs baseline: 1.0146x; 1.0146x over previous
import numpy as np
import jax
import jax.numpy as jnp
from jax import lax
from jax.experimental import pallas as pl
from jax.experimental.pallas import tpu as pltpu
from jax.experimental.pallas import tpu_sc as plsc

_B = 256
_LMAX = 256
_M = 2 * _LMAX - 1
_K = _LMAX * _LMAX
_NC, _NS, _L = 2, 16, 16
_NW = _NC * _NS


def _cost_partition():
    cols = np.arange(_M)
    starts = np.abs(cols - (_LMAX - 1))
    l_of_k = np.concatenate([np.arange(s, _LMAX) for s in starts])
    c_of_k = np.repeat(cols, _LMAX - starts)

    def spans_for(target):
        spans, ks = [], 0
        while ks < _K and len(spans) < _NW:
            c0 = int(c_of_k[ks])
            ke = ks + 16
            while ke < _K:
                nxt = min(ke + 16, _K)
                w = int(c_of_k[nxt - 1]) - c0 + 1
                if 256 * w + (nxt - ks) > target:
                    break
                ke = nxt
            spans.append((ks, ke))
            ks = ke
        return spans, ks

    lo, hi = 4096, 32768
    while lo < hi:
        mid = (lo + hi) // 2
        _, ks = spans_for(mid)
        if ks >= _K:
            hi = mid
        else:
            lo = mid + 1
    spans, ks = spans_for(lo)
    assert ks == _K and len(spans) <= _NW
    while len(spans) < _NW:
        i = max(range(len(spans)), key=lambda j: spans[j][1] - spans[j][0])
        a, b = spans[i]
        m = a + ((b - a) // 32) * 16
        spans[i:i + 1] = [(a, m), (m, b)]
    return spans, l_of_k, c_of_k


def _build_geometry():
    spans, l_of_k, c_of_k = _cost_partition()
    raw = []
    for ks, ke in spans:
        c0, c1 = int(c_of_k[ks]), int(c_of_k[ke - 1])
        raw.append((ks, ke - ks, c0, c1 - c0 + 1))

    order = sorted(range(_NW), key=lambda s: (raw[s][1], raw[s][3]))
    classes = []
    group = []
    for s in order:
        group.append(s)
        if len(group) == 4 or s == order[-1]:
            ln_c = max(raw[g][1] for g in group)
            mems = []
            w_need = 0
            for g in group:
                k0 = min(raw[g][0], _K - ln_c)
                c0 = int(c_of_k[k0])
                c1 = int(c_of_k[k0 + ln_c - 1])
                w_need = max(w_need, c1 - c0 + 1)
                mems.append((g, k0, c0))
            w_c = w_need
            offs = [(g, k0, min(c0, _M - w_c)) for g, k0, c0 in mems]
            classes.append((w_c, ln_c, offs))
            group = []

    ln_max = max(ln for _, ln, _ in classes)
    packed = np.zeros((_NW, ln_max), np.int32)
    for w_c, ln_c, offs in classes:
        for s, k0, c0c in offs:
            lk = l_of_k[k0:k0 + ln_c]
            ck = c_of_k[k0:k0 + ln_c]
            assert c0c >= 0 and ck.max() < c0c + w_c and ck.min() >= c0c
            packed[s, :ln_c] = (((ck - c0c).astype(np.int32) << 16)
                                | lk.astype(np.int32))
    return classes, packed, ln_max


_CLASSES, _PACKED, _LN_MAX = _build_geometry()


def _sc_body(tensor_hbm, idx_hbm, out_hbm, idx_v, ob0, ob1, isem0, isem1,
             osem0, osem1):
    wid = lax.axis_index("c") * _NS + lax.axis_index("s")
    pltpu.sync_copy(idx_hbm.at[wid], idx_v)

    for w_c, ln_c, offs in _CLASSES:
        if len(offs) == 1:
            s, k0v, c0c = offs[0]
            is_member = wid == s
        else:
            is_member = jnp.bool_(False)
            c0c = jnp.int32(0)
            k0v = jnp.int32(0)
            for s, k0, c0 in offs:
                hit = wid == s
                is_member = jnp.logical_or(is_member, hit)
                c0c = jnp.where(hit, jnp.int32(c0), c0c)
                k0v = jnp.where(hit, jnp.int32(k0), k0v)
            k0v = pl.multiple_of(k0v, 16)

        @pl.when(is_member)
        def _cls(w_c=w_c, ln_c=ln_c, c0c=c0c, k0v=k0v):
            ng = ln_c // _L

            def gather(slab, ob):
                @plsc.parallel_loop(0, ng, unroll=8)
                def _g(g):
                    iv = idx_v[pl.ds(g * _L, _L)]
                    rows = lax.shift_right_logical(iv, 16)
                    cls_ = lax.bitwise_and(iv, jnp.int32(0xFFFF))
                    ob[pl.ds(g * _L, _L)] = plsc.load_gather(
                        slab, [rows, cls_])

            def scoped(slab0, slab1):
                def in_copy(b, slab, sem):
                    return pltpu.make_async_copy(
                        tensor_hbm.at[b, pl.ds(c0c, w_c), :], slab, sem)

                def out_copy(b, ob, sem):
                    return pltpu.make_async_copy(
                        ob.at[pl.ds(0, ln_c)],
                        out_hbm.at[b, pl.ds(k0v, ln_c)], sem)

                in_copy(0, slab0, isem0).start()
                in_copy(1, slab1, isem1).start()

                @pl.loop(0, _B // 2)
                def _bb(bb):
                    b0 = bb * 2
                    b1 = b0 + 1

                    @pl.when(bb > 0)
                    def _():
                        out_copy(b0 - 2, ob0, osem0).wait()
                    in_copy(b0, slab0, isem0).wait()
                    gather(slab0, ob0)

                    @pl.when(bb < _B // 2 - 1)
                    def _():
                        in_copy(b0 + 2, slab0, isem0).start()
                    out_copy(b0, ob0, osem0).start()

                    @pl.when(bb > 0)
                    def _():
                        out_copy(b1 - 2, ob1, osem1).wait()
                    in_copy(b1, slab1, isem1).wait()
                    gather(slab1, ob1)

                    @pl.when(bb < _B // 2 - 1)
                    def _():
                        in_copy(b1 + 2, slab1, isem1).start()
                    out_copy(b1, ob1, osem1).start()

                out_copy(_B - 2, ob0, osem0).wait()
                out_copy(_B - 1, ob1, osem1).wait()

            pl.run_scoped(
                scoped,
                pltpu.VMEM((w_c, _LMAX), jnp.float32),
                pltpu.VMEM((w_c, _LMAX), jnp.float32),
            )


def kernel(tensor):
    tensor_t = jnp.swapaxes(tensor, 1, 2)
    idx = jnp.asarray(_PACKED)
    mesh = plsc.VectorSubcoreMesh(core_axis_name="c", subcore_axis_name="s")
    f = pl.kernel(
        _sc_body,
        out_type=jax.ShapeDtypeStruct((_B, _K), jnp.float32),
        mesh=mesh,
        compiler_params=pltpu.CompilerParams(
            use_tc_tiling_on_sc=False, needs_layout_passes=False),
        scratch_types=[
            pltpu.VMEM((_LN_MAX,), jnp.int32),
            pltpu.VMEM((_LN_MAX,), jnp.float32),
            pltpu.VMEM((_LN_MAX,), jnp.float32),
            pltpu.SemaphoreType.DMA,
            pltpu.SemaphoreType.DMA,
            pltpu.SemaphoreType.DMA,
            pltpu.SemaphoreType.DMA,
        ],
    )
    return f(tensor_t, idx)


# speedup vs baseline: 1.0502x; 1.0352x over previous
import numpy as np
import jax
import jax.numpy as jnp
from jax import lax
from jax.experimental import pallas as pl
from jax.experimental.pallas import tpu as pltpu
from jax.experimental.pallas import tpu_sc as plsc

_B = 256
_LMAX = 256
_M = 2 * _LMAX - 1
_K = _LMAX * _LMAX
_NC, _NS, _L = 2, 16, 16
_NW = _NC * _NS


def _cost_partition():
    cols = np.arange(_M)
    starts = np.abs(cols - (_LMAX - 1))
    l_of_k = np.concatenate([np.arange(s, _LMAX) for s in starts])
    c_of_k = np.repeat(cols, _LMAX - starts)

    def spans_for(target):
        spans, ks = [], 0
        while ks < _K and len(spans) < _NW:
            c0 = int(c_of_k[ks])
            ke = ks + 128
            while ke < _K:
                nxt = min(ke + 128, _K)
                w = int(c_of_k[nxt - 1]) - c0 + 1
                if 256 * w + (nxt - ks) > target:
                    break
                ke = nxt
            spans.append((ks, ke))
            ks = ke
        return spans, ks

    lo, hi = 4096, 32768
    while lo < hi:
        mid = (lo + hi) // 2
        _, ks = spans_for(mid)
        if ks >= _K:
            hi = mid
        else:
            lo = mid + 1
    spans, ks = spans_for(lo)
    assert ks == _K and len(spans) <= _NW
    while len(spans) < _NW:
        i = max(range(len(spans)), key=lambda j: spans[j][1] - spans[j][0])
        a, b = spans[i]
        m = a + max((b - a) // 256, 1) * 128
        spans[i:i + 1] = [(a, m), (m, b)]
    return spans, l_of_k, c_of_k


def _build_geometry():
    spans, l_of_k, c_of_k = _cost_partition()
    raw = []
    for ks, ke in spans:
        c0, c1 = int(c_of_k[ks]), int(c_of_k[ke - 1])
        raw.append((ks, ke - ks, c0, c1 - c0 + 1))

    order = sorted(range(_NW), key=lambda s: (raw[s][1], raw[s][3]))
    classes = []
    group = []
    for s in order:
        group.append(s)
        if len(group) == 4 or s == order[-1]:
            ln_c = max(raw[g][1] for g in group)
            mems = []
            w_need = 0
            for g in group:
                k0 = min(raw[g][0], _K - ln_c)
                c0 = int(c_of_k[k0])
                c1 = int(c_of_k[k0 + ln_c - 1])
                w_need = max(w_need, c1 - c0 + 1)
                mems.append((g, k0, c0))
            w_c = w_need
            offs = [(g, k0, min(c0, _M - w_c)) for g, k0, c0 in mems]
            classes.append((w_c, ln_c, offs))
            group = []

    ln_max = max(ln for _, ln, _ in classes)
    packed = np.zeros((_NW, ln_max), np.int32)
    for w_c, ln_c, offs in classes:
        for s, k0, c0c in offs:
            lk = l_of_k[k0:k0 + ln_c]
            ck = c_of_k[k0:k0 + ln_c]
            assert c0c >= 0 and ck.max() < c0c + w_c and ck.min() >= c0c
            packed[s, :ln_c] = (((ck - c0c).astype(np.int32) << 16)
                                | lk.astype(np.int32))
    return classes, packed, ln_max


_CLASSES, _PACKED, _LN_MAX = _build_geometry()


def _sc_body(tensor_hbm, idx_hbm, out_hbm, idx_v, ob0, ob1, isem0, isem1,
             osem0, osem1):
    wid = lax.axis_index("c") * _NS + lax.axis_index("s")
    pltpu.sync_copy(idx_hbm.at[wid], idx_v)

    for w_c, ln_c, offs in _CLASSES:
        if len(offs) == 1:
            s, k0v, c0c = offs[0]
            is_member = wid == s
        else:
            is_member = jnp.bool_(False)
            c0c = jnp.int32(0)
            k0v = jnp.int32(0)
            for s, k0, c0 in offs:
                hit = wid == s
                is_member = jnp.logical_or(is_member, hit)
                c0c = jnp.where(hit, jnp.int32(c0), c0c)
                k0v = jnp.where(hit, jnp.int32(k0), k0v)
            k0v = pl.multiple_of(k0v, 128)

        @pl.when(is_member)
        def _cls(w_c=w_c, ln_c=ln_c, c0c=c0c, k0v=k0v):
            ng = ln_c // _L

            def gather(slab, ob):
                @plsc.parallel_loop(0, ng, unroll=8)
                def _g(g):
                    iv = idx_v[pl.ds(g * _L, _L)]
                    rows = lax.shift_right_logical(iv, 16)
                    cls_ = lax.bitwise_and(iv, jnp.int32(0xFFFF))
                    ob[g // 8, pl.ds((g % 8) * _L, _L)] = plsc.load_gather(
                        slab, [rows, cls_])

            def scoped(slab0, slab1):
                def in_copy(b, slab, sem):
                    return pltpu.make_async_copy(
                        tensor_hbm.at[b, pl.ds(c0c, w_c), :], slab, sem)

                def out_copy(b, ob, sem):
                    lane = pl.multiple_of((b % 8) * 128, 128)
                    return pltpu.make_async_copy(
                        ob.at[pl.ds(0, ln_c // 128), :],
                        out_hbm.at[b // 8, pl.ds(k0v // 128, ln_c // 128),
                                   pl.ds(lane, 128)], sem)

                in_copy(0, slab0, isem0).start()
                in_copy(1, slab1, isem1).start()

                @pl.loop(0, _B // 2)
                def _bb(bb):
                    b0 = bb * 2
                    b1 = b0 + 1

                    @pl.when(bb > 0)
                    def _():
                        out_copy(b0 - 2, ob0, osem0).wait()
                    in_copy(b0, slab0, isem0).wait()
                    gather(slab0, ob0)

                    @pl.when(bb < _B // 2 - 1)
                    def _():
                        in_copy(b0 + 2, slab0, isem0).start()
                    out_copy(b0, ob0, osem0).start()

                    @pl.when(bb > 0)
                    def _():
                        out_copy(b1 - 2, ob1, osem1).wait()
                    in_copy(b1, slab1, isem1).wait()
                    gather(slab1, ob1)

                    @pl.when(bb < _B // 2 - 1)
                    def _():
                        in_copy(b1 + 2, slab1, isem1).start()
                    out_copy(b1, ob1, osem1).start()

                out_copy(_B - 2, ob0, osem0).wait()
                out_copy(_B - 1, ob1, osem1).wait()

            pl.run_scoped(
                scoped,
                pltpu.VMEM((w_c, _LMAX), jnp.float32),
                pltpu.VMEM((w_c, _LMAX), jnp.float32),
            )


def kernel(tensor):
    tensor_t = jnp.swapaxes(tensor, 1, 2)
    idx = jnp.asarray(_PACKED)
    mesh = plsc.VectorSubcoreMesh(core_axis_name="c", subcore_axis_name="s")
    f = pl.kernel(
        _sc_body,
        out_type=jax.ShapeDtypeStruct((_B // 8, _K // 128, 8 * 128),
                                      jnp.float32),
        mesh=mesh,
        compiler_params=pltpu.CompilerParams(
            use_tc_tiling_on_sc=False, needs_layout_passes=False),
        scratch_types=[
            pltpu.VMEM((_LN_MAX,), jnp.int32),
            pltpu.VMEM((_LN_MAX // 128, 128), jnp.float32),
            pltpu.VMEM((_LN_MAX // 128, 128), jnp.float32),
            pltpu.SemaphoreType.DMA,
            pltpu.SemaphoreType.DMA,
            pltpu.SemaphoreType.DMA,
            pltpu.SemaphoreType.DMA,
        ],
    )
    out3 = f(tensor_t, idx)
    return (out3.reshape(_B // 8, _K // 128, 8, 128)
            .transpose(0, 2, 1, 3).reshape(_B, _K))


# speedup vs baseline: 1.1642x; 1.1085x over previous
import numpy as np
import jax
import jax.numpy as jnp
from jax import lax
from jax.experimental import pallas as pl
from jax.experimental.pallas import tpu as pltpu
from jax.experimental.pallas import tpu_sc as plsc

_B = 256
_LMAX = 256
_M = 2 * _LMAX - 1
_K = _LMAX * _LMAX
_NC, _NS, _L = 2, 16, 16
_NW = _NC * _NS


def _cost_partition():
    cols = np.arange(_M)
    starts = np.abs(cols - (_LMAX - 1))
    l_of_k = np.concatenate([np.arange(s, _LMAX) for s in starts])
    c_of_k = np.repeat(cols, _LMAX - starts)

    def spans_for(target):
        spans, ks = [], 0
        while ks < _K and len(spans) < _NW:
            c0 = int(c_of_k[ks])
            ke = ks + 128
            while ke < _K:
                nxt = min(ke + 128, _K)
                w = int(c_of_k[nxt - 1]) - c0 + 1
                if 256 * w + (nxt - ks) > target:
                    break
                ke = nxt
            spans.append((ks, ke))
            ks = ke
        return spans, ks

    lo, hi = 4096, 32768
    while lo < hi:
        mid = (lo + hi) // 2
        _, ks = spans_for(mid)
        if ks >= _K:
            hi = mid
        else:
            lo = mid + 1
    spans, ks = spans_for(lo)
    assert ks == _K and len(spans) <= _NW
    while len(spans) < _NW:
        i = max(range(len(spans)), key=lambda j: spans[j][1] - spans[j][0])
        a, b = spans[i]
        m = a + max((b - a) // 256, 1) * 128
        spans[i:i + 1] = [(a, m), (m, b)]
    return spans, l_of_k, c_of_k


def _build_geometry():
    spans, l_of_k, c_of_k = _cost_partition()
    raw = []
    for ks, ke in spans:
        c0, c1 = int(c_of_k[ks]), int(c_of_k[ke - 1])
        raw.append((ks, ke - ks, c0, c1 - c0 + 1))

    order = sorted(range(_NW), key=lambda s: (raw[s][1], raw[s][3]))
    classes = []
    group = []
    for s in order:
        group.append(s)
        if len(group) == 2 or s == order[-1]:
            ln_c = max(raw[g][1] for g in group)
            mems = []
            w_need = 0
            for g in group:
                k0 = min(raw[g][0], _K - ln_c)
                c0 = int(c_of_k[k0])
                c1 = int(c_of_k[k0 + ln_c - 1])
                w_need = max(w_need, c1 - c0 + 1)
                mems.append((g, k0, c0))
            w_c = w_need
            offs = [(g, k0, min(c0, _M - w_c)) for g, k0, c0 in mems]
            classes.append((w_c, ln_c, offs))
            group = []

    ln_max = max(ln for _, ln, _ in classes)
    packed = np.zeros((_NW, ln_max), np.int32)
    for w_c, ln_c, offs in classes:
        for s, k0, c0c in offs:
            lk = l_of_k[k0:k0 + ln_c]
            ck = c_of_k[k0:k0 + ln_c]
            assert c0c >= 0 and ck.max() < c0c + w_c and ck.min() >= c0c
            packed[s, :ln_c] = (((ck - c0c).astype(np.int32) << 16)
                                | lk.astype(np.int32))
    return classes, packed, ln_max


_CLASSES, _PACKED, _LN_MAX = _build_geometry()


def _sc_body(tensor_hbm, idx_hbm, out_hbm, idx_v, ob0, ob1, isem0, isem1,
             osem0, osem1):
    wid = lax.axis_index("c") * _NS + lax.axis_index("s")
    pltpu.sync_copy(idx_hbm.at[wid], idx_v)

    for w_c, ln_c, offs in _CLASSES:
        if len(offs) == 1:
            s, k0v, c0c = offs[0]
            is_member = wid == s
        else:
            is_member = jnp.bool_(False)
            c0c = jnp.int32(0)
            k0v = jnp.int32(0)
            for s, k0, c0 in offs:
                hit = wid == s
                is_member = jnp.logical_or(is_member, hit)
                c0c = jnp.where(hit, jnp.int32(c0), c0c)
                k0v = jnp.where(hit, jnp.int32(k0), k0v)
            k0v = pl.multiple_of(k0v, 128)

        @pl.when(is_member)
        def _cls(w_c=w_c, ln_c=ln_c, c0c=c0c, k0v=k0v):
            ng = ln_c // _L

            def gather(slab, ob):
                @plsc.parallel_loop(0, ng, unroll=8)
                def _g(g):
                    iv = idx_v[pl.ds(g * _L, _L)]
                    rows = lax.shift_right_logical(iv, 16)
                    cls_ = lax.bitwise_and(iv, jnp.int32(0xFFFF))
                    ob[g // 8, pl.ds((g % 8) * _L, _L)] = plsc.load_gather(
                        slab, [rows, cls_])

            def scoped(slab0, slab1):
                def in_copy(b, slab, sem):
                    return pltpu.make_async_copy(
                        tensor_hbm.at[b, pl.ds(c0c, w_c), :], slab, sem)

                def out_copy(b, ob, sem):
                    lane = pl.multiple_of((b % 8) * 128, 128)
                    return pltpu.make_async_copy(
                        ob.at[pl.ds(0, ln_c // 128), :],
                        out_hbm.at[b // 8, pl.ds(k0v // 128, ln_c // 128),
                                   pl.ds(lane, 128)], sem)

                in_copy(0, slab0, isem0).start()
                in_copy(1, slab1, isem1).start()

                @pl.loop(0, _B // 2)
                def _bb(bb):
                    b0 = bb * 2
                    b1 = b0 + 1

                    @pl.when(bb > 0)
                    def _():
                        out_copy(b0 - 2, ob0, osem0).wait()
                    in_copy(b0, slab0, isem0).wait()
                    gather(slab0, ob0)

                    @pl.when(bb < _B // 2 - 1)
                    def _():
                        in_copy(b0 + 2, slab0, isem0).start()
                    out_copy(b0, ob0, osem0).start()

                    @pl.when(bb > 0)
                    def _():
                        out_copy(b1 - 2, ob1, osem1).wait()
                    in_copy(b1, slab1, isem1).wait()
                    gather(slab1, ob1)

                    @pl.when(bb < _B // 2 - 1)
                    def _():
                        in_copy(b1 + 2, slab1, isem1).start()
                    out_copy(b1, ob1, osem1).start()

                out_copy(_B - 2, ob0, osem0).wait()
                out_copy(_B - 1, ob1, osem1).wait()

            pl.run_scoped(
                scoped,
                pltpu.VMEM((w_c, _LMAX), jnp.float32),
                pltpu.VMEM((w_c, _LMAX), jnp.float32),
            )


def kernel(tensor):
    tensor_t = jnp.swapaxes(tensor, 1, 2)
    idx = jnp.asarray(_PACKED)
    mesh = plsc.VectorSubcoreMesh(core_axis_name="c", subcore_axis_name="s")
    f = pl.kernel(
        _sc_body,
        out_type=jax.ShapeDtypeStruct((_B // 8, _K // 128, 8 * 128),
                                      jnp.float32),
        mesh=mesh,
        compiler_params=pltpu.CompilerParams(
            use_tc_tiling_on_sc=False, needs_layout_passes=False),
        scratch_types=[
            pltpu.VMEM((_LN_MAX,), jnp.int32),
            pltpu.VMEM((_LN_MAX // 128, 128), jnp.float32),
            pltpu.VMEM((_LN_MAX // 128, 128), jnp.float32),
            pltpu.SemaphoreType.DMA,
            pltpu.SemaphoreType.DMA,
            pltpu.SemaphoreType.DMA,
            pltpu.SemaphoreType.DMA,
        ],
    )
    out3 = f(tensor_t, idx)
    return (out3.reshape(_B // 8, _K // 128, 8, 128)
            .transpose(0, 2, 1, 3).reshape(_B, _K))
